# baseline (speedup 1.0000x reference)
import functools

import jax
import jax.numpy as jnp
from jax import lax
from jax.experimental import pallas as pl
from jax.experimental.pallas import tpu as pltpu
from jax.experimental.pallas import tpu_sc as plsc

_PERIOD = 4096
_ROWS = 16384
_D = 2048
_NC = 2
_NS = 16
_NW = _NC * _NS
_ROWS_PER_W = _ROWS // _NW
_W_PER_GROUP = _PERIOD // _ROWS_PER_W
_B = 8
_NBUF = 8
_NVM = 4
_INFL = 4
_STEPS = _ROWS_PER_W // _B
_NGRP = _STEPS // _NBUF


@functools.partial(
    pl.kernel,
    mesh=plsc.VectorSubcoreMesh(core_axis_name="c", subcore_axis_name="s"),
    out_type=jax.ShapeDtypeStruct((_ROWS // _PERIOD, _PERIOD, _D), jnp.float32),
    scratch_types=(
        [pltpu.VMEM((_NVM, _B, _D), jnp.float32)]
        + [pltpu.VMEM_SHARED((_NS, _NBUF - _NVM, _B, _D), jnp.float32)]
        + [pltpu.SemaphoreType.DMA for _ in range(2 * _NBUF)]
    ),
)
def _gather_view(x_hbm, out_hbm, vbuf, shared, *sems):
    rsems = sems[:_NBUF]
    wsems = sems[_NBUF:]
    s = lax.axis_index("s")
    wid = s * _NC + lax.axis_index("c")
    g = wid // _W_PER_GROUP
    off = (wid % _W_PER_GROUP) * _ROWS_PER_W
    base = wid * _ROWS_PER_W

    def buf(b):
        return vbuf.at[b] if b < _NVM else shared.at[s, b - _NVM]

    def read(i, b):
        return pltpu.make_async_copy(
            x_hbm.at[pl.ds(base + i * _B, _B)], buf(b), rsems[b])

    def write(i, b):
        return pltpu.make_async_copy(
            buf(b), out_hbm.at[g, pl.ds(off + i * _B, _B)], wsems[b])

    for b in range(_INFL):
        read(b, b).start()

    for i in range(_NBUF):
        read(i, i).wait()
        write(i, i).start()
        if i >= _INFL:
            write(i - _INFL, i - _INFL).wait()
        read(i + _INFL, (i + _INFL) % _NBUF).start()

    @pl.loop(1, _NGRP - 1)
    def _loop(t):
        i0 = t * _NBUF
        for b in range(_NBUF):
            i = i0 + b
            read(i, b).wait()
            write(i, b).start()
            write(i, (b + _INFL) % _NBUF).wait()
            read(i + _INFL, (b + _INFL) % _NBUF).start()

    i0 = (_NGRP - 1) * _NBUF
    for b in range(_NBUF):
        i = i0 + b
        read(i, b).wait()
        write(i, b).start()
        write(i - _INFL, (b + _INFL) % _NBUF).wait()
        if b < _INFL:
            read(i + _INFL, (b + _INFL) % _NBUF).start()
    for b in range(_NBUF - _INFL, _NBUF):
        write(i0 + b, b).wait()


def kernel(x):
    return _gather_view(x)


# speedup vs baseline: 1.0149x; 1.0149x over previous
import functools

import jax
import jax.numpy as jnp
from jax import lax
from jax.experimental import pallas as pl
from jax.experimental.pallas import tpu as pltpu
from jax.experimental.pallas import tpu_sc as plsc

_PERIOD = 4096
_ROWS = 16384
_D = 2048
_NC = 2
_NS = 16
_NW = _NC * _NS
_ROWS_PER_W = _ROWS // _NW
_W_PER_GROUP = _PERIOD // _ROWS_PER_W
_B = 8
_NBUF = 4
_STEPS = _ROWS_PER_W // _B
_NGRP = _STEPS // _NBUF


@functools.partial(
    pl.kernel,
    mesh=plsc.VectorSubcoreMesh(core_axis_name="c", subcore_axis_name="s"),
    out_type=jax.ShapeDtypeStruct((_ROWS // _PERIOD, _PERIOD, _D), jnp.float32),
    scratch_types=(
        [pltpu.VMEM_SHARED((_NS, _NBUF, _B, _D), jnp.float32)]
        + [pltpu.SemaphoreType.DMA for _ in range(2 * _NBUF)]
    ),
)
def _gather_view(x_hbm, out_hbm, shared, *sems):
    rsems = sems[:_NBUF]
    wsems = sems[_NBUF:]
    s = lax.axis_index("s")
    wid = s * _NC + lax.axis_index("c")
    g = wid // _W_PER_GROUP
    off = (wid % _W_PER_GROUP) * _ROWS_PER_W
    base = wid * _ROWS_PER_W

    def read(i, b):
        return pltpu.make_async_copy(
            x_hbm.at[pl.ds(base + i * _B, _B)], shared.at[s, b], rsems[b])

    def write(i, b):
        return pltpu.make_async_copy(
            shared.at[s, b], out_hbm.at[g, pl.ds(off + i * _B, _B)], wsems[b])

    read(0, 0).start()
    read(1, 1).start()

    read(0, 0).wait(); write(0, 0).start(); read(2, 2).start()
    read(1, 1).wait(); write(1, 1).start(); read(3, 3).start()
    read(2, 2).wait(); write(2, 2).start(); write(0, 0).wait(); read(4, 0).start()
    read(3, 3).wait(); write(3, 3).start(); write(1, 1).wait(); read(5, 1).start()

    @pl.loop(1, _NGRP - 1)
    def _loop(t):
        i0 = t * _NBUF
        for b in range(_NBUF):
            i = i0 + b
            read(i, b).wait()
            write(i, b).start()
            write(i, (b + 2) % _NBUF).wait()
            read(i + 2, (b + 2) % _NBUF).start()

    i0 = (_NGRP - 1) * _NBUF
    read(i0 + 0, 0).wait(); write(i0 + 0, 0).start(); write(i0 - 2, 2).wait(); read(i0 + 2, 2).start()
    read(i0 + 1, 1).wait(); write(i0 + 1, 1).start(); write(i0 - 1, 3).wait(); read(i0 + 3, 3).start()
    read(i0 + 2, 2).wait(); write(i0 + 2, 2).start(); write(i0 + 0, 0).wait()
    read(i0 + 3, 3).wait(); write(i0 + 3, 3).start(); write(i0 + 1, 1).wait()
    write(i0 + 2, 2).wait()
    write(i0 + 3, 3).wait()


def kernel(x):
    return _gather_view(x)
